# baseline (speedup 1.0000x reference)
import functools

import jax
import jax.numpy as jnp
import numpy as np
from jax import lax
from jax.experimental import pallas as pl
from jax.experimental.pallas import tpu as pltpu
from jax.experimental.pallas import tpu_sc as plsc

N = 10000
E = 320000
F_IN = 128
F_HID = 128
F_OUT = 40
F_OUT_PAD = 64

NC = 2
NS = 16
NW = NC * NS
E_T = E // NW
C = 80
NCH = E_T // C
R_T = 624
R_REM = N - NS * R_T


def _spmm_kernel_body(F, idx_hbm, feats_hbm, out_hbm,
                      idx_v0, idx_v1, rows0, rows1, msgs,
                      gsem0, gsem1, stsem0, stsem1, acc):
    nf = F // 16
    cid = lax.axis_index("c")
    sid = lax.axis_index("s")
    widg = cid * NS + sid
    bufs = ((idx_v0, rows0, gsem0, stsem0),
            (idx_v1, rows1, gsem1, stsem1))

    zero = jnp.zeros((16,), jnp.float32)

    def zrow(r, _):
        for f in range(nf):
            msgs[r, pl.ds(f * 16, 16)] = zero
        return _

    lax.fori_loop(0, C, zrow, 0, unroll=4)
    for k in range(R_T // C):
        pltpu.sync_copy(msgs, acc.at[pl.ds(sid * R_T + k * C, C)])
    rem = R_T - (R_T // C) * C
    pltpu.sync_copy(msgs.at[pl.ds(0, rem)],
                    acc.at[pl.ds(sid * R_T + R_T - rem, rem)])

    @pl.when(sid == NS - 1)
    def _():
        pltpu.sync_copy(msgs.at[pl.ds(0, R_REM)],
                        acc.at[pl.ds(NS * R_T, R_REM)])

    plsc.subcore_barrier()

    def stage_start(j, b):
        idx_v, _, _, stsem = bufs[b]
        pltpu.async_copy(idx_hbm.at[:, widg, j], idx_v, stsem)

    def stage_wait(b):
        idx_v, _, _, stsem = bufs[b]
        pltpu.make_async_copy(idx_hbm.at[:, widg, 0], idx_v, stsem).wait()

    def gather_start(b):
        idx_v, rows, gsem, _ = bufs[b]
        pltpu.async_copy(feats_hbm.at[idx_v.at[1]], rows, gsem)

    def gather_wait(b):
        idx_v, rows, gsem, _ = bufs[b]
        pltpu.make_async_copy(feats_hbm.at[idx_v.at[1]], rows, gsem).wait()

    himask = jnp.full((16,), -65536, jnp.int32)

    def compute_scatter(b):
        idx_v, rows, _, _ = bufs[b]

        @plsc.parallel_loop(0, C, unroll=8)
        def edge(e):
            wb = plsc.bitcast(
                plsc.load_gather(idx_v.at[2],
                                 [jnp.full((16,), e, jnp.int32)]),
                jnp.float32)
            for f in range(F // 32):
                v = rows[e, pl.ds(f * 16, 16)]
                lo = plsc.bitcast(v << 16, jnp.float32)
                hi = plsc.bitcast(v & himask, jnp.float32)
                msgs[e, pl.ds(f * 32, 16)] = lo * wb
                msgs[e, pl.ds(f * 32 + 16, 16)] = hi * wb

        pltpu.sync_copy(msgs, acc.at[idx_v.at[0]], add=True)

    stage_start(0, 0)
    stage_start(1, 1)
    stage_wait(0)
    gather_start(0)

    def pair(g, carry):
        j0 = 2 * g
        stage_wait(1)
        gather_start(1)
        gather_wait(0)
        compute_scatter(0)
        stage_start(j0 + 2, 0)
        stage_wait(0)
        gather_start(0)
        gather_wait(1)
        compute_scatter(1)

        @pl.when(j0 + 3 < NCH)
        def _stage_next():
            stage_start(j0 + 3, 1)

        return carry

    lax.fori_loop(0, NCH // 2, pair, 0)

    gather_wait(0)
    compute_scatter(0)

    plsc.subcore_barrier()

    pltpu.sync_copy(acc.at[pl.ds(sid * R_T, R_T)],
                    out_hbm.at[cid, pl.ds(sid * R_T, R_T)])

    @pl.when(sid == NS - 1)
    def _():
        pltpu.sync_copy(acc.at[pl.ds(NS * R_T, R_REM)],
                        out_hbm.at[cid, pl.ds(NS * R_T, R_REM)])


@functools.lru_cache(maxsize=None)
def _make_spmm(F):
    mesh = plsc.VectorSubcoreMesh(core_axis_name="c", subcore_axis_name="s",
                                  num_cores=NC, num_subcores=NS)
    return pl.kernel(
        functools.partial(_spmm_kernel_body, F),
        mesh=mesh,
        out_type=jax.ShapeDtypeStruct((NC, N, F), jnp.float32),
        scratch_types=[
            pltpu.VMEM((3, C), jnp.int32),
            pltpu.VMEM((3, C), jnp.int32),
            pltpu.VMEM((C, F // 2), jnp.int32),
            pltpu.VMEM((C, F // 2), jnp.int32),
            pltpu.VMEM((C, F), jnp.float32),
            pltpu.SemaphoreType.DMA,
            pltpu.SemaphoreType.DMA,
            pltpu.SemaphoreType.DMA,
            pltpu.SemaphoreType.DMA,
            pltpu.VMEM_SHARED((N, F), jnp.float32),
        ],
        compiler_params=pltpu.CompilerParams(use_tc_tiling_on_sc=False,
                                             needs_layout_passes=False),
    )


_RB = 1000


def _pack_perm(fb):
    half = fb // 2
    p = np.empty(fb, np.int32)
    for j in range(half):
        p[j] = (j // 16) * 32 + j % 16
        p[half + j] = (j // 16) * 32 + 16 + j % 16
    return p


_PERM_HID = _pack_perm(F_HID)
_PERM_OUT = _pack_perm(F_OUT_PAD)


def _pack_words(y):
    half = y.shape[-1] // 2
    u = lax.bitcast_convert_type(y, jnp.uint32)
    r = u + jnp.uint32(0x7FFF) + ((u >> 16) & jnp.uint32(1))
    lo = r[:, :half] >> 16
    hi = r[:, half:] & jnp.uint32(0xFFFF0000)
    return lax.bitcast_convert_type(lo | hi, jnp.int32)


def _mm_body(x_ref, w_ref, o_ref):
    o_ref[...] = _pack_words(jnp.dot(x_ref[...], w_ref[...],
                                     preferred_element_type=jnp.float32))


def _fuse_body(p_ref, b_ref, w_ref, o_ref):
    h = jnp.maximum(p_ref[0] + p_ref[1] + b_ref[...], 0.0)
    o_ref[...] = _pack_words(jnp.dot(h, w_ref[...],
                                     preferred_element_type=jnp.float32))


def _comb_body(q_ref, b_ref, o_ref):
    s = q_ref[0] + q_ref[1]
    o_ref[...] = s[:, :F_OUT] + b_ref[...]


def _mm(x, w):
    return pl.pallas_call(
        _mm_body,
        grid=(N // _RB,),
        in_specs=[pl.BlockSpec((_RB, F_IN), lambda i: (i, 0)),
                  pl.BlockSpec((F_IN, F_HID), lambda i: (0, 0))],
        out_specs=pl.BlockSpec((_RB, F_HID // 2), lambda i: (i, 0)),
        out_shape=jax.ShapeDtypeStruct((N, F_HID // 2), jnp.int32),
    )(x, w)


def _fuse(p, b1, w2p):
    return pl.pallas_call(
        _fuse_body,
        grid=(N // _RB,),
        in_specs=[pl.BlockSpec((NC, _RB, F_HID), lambda i: (0, i, 0)),
                  pl.BlockSpec((1, F_HID), lambda i: (0, 0)),
                  pl.BlockSpec((F_HID, F_OUT_PAD), lambda i: (0, 0))],
        out_specs=pl.BlockSpec((_RB, F_OUT_PAD // 2), lambda i: (i, 0)),
        out_shape=jax.ShapeDtypeStruct((N, F_OUT_PAD // 2), jnp.int32),
    )(p, b1, w2p)


def _combine(q, b2):
    return pl.pallas_call(
        _comb_body,
        grid=(N // _RB,),
        in_specs=[pl.BlockSpec((NC, _RB, F_OUT_PAD), lambda i: (0, i, 0)),
                  pl.BlockSpec((1, F_OUT), lambda i: (0, 0))],
        out_specs=pl.BlockSpec((_RB, F_OUT), lambda i: (i, 0)),
        out_shape=jax.ShapeDtypeStruct((N, F_OUT), jnp.float32),
    )(q, b2)


def kernel(x, edge_index, edge_weight, W1, b1, W2, b2):
    wbits = lax.bitcast_convert_type(edge_weight.astype(jnp.float32),
                                     jnp.int32)
    idx = jnp.concatenate(
        [edge_index.astype(jnp.int32).reshape(2, NW, NCH, C),
         wbits.reshape(1, NW, NCH, C)], axis=0)
    w2p = jnp.pad(W2, ((0, 0), (0, F_OUT_PAD - F_OUT)))[:, _PERM_OUT]

    xw1 = _mm(x, W1[:, _PERM_HID])
    p = _make_spmm(F_HID)(idx, xw1)
    hw2 = _fuse(p, b1.reshape(1, F_HID), w2p)
    q = _make_spmm(F_OUT_PAD)(idx, hw2)
    return _combine(q, b2.reshape(1, F_OUT))


# speedup vs baseline: 1.0427x; 1.0427x over previous
import functools

import jax
import jax.numpy as jnp
import numpy as np
from jax import lax
from jax.experimental import pallas as pl
from jax.experimental.pallas import tpu as pltpu
from jax.experimental.pallas import tpu_sc as plsc

N = 10000
E = 320000
F_IN = 128
F_HID = 128
F_OUT = 40
F_OUT_PAD = 64

NC = 2
NS = 16
NW = NC * NS
E_T = E // NW
C = 80
NCH = E_T // C
R_T = 624
R_REM = N - NS * R_T


def _spmm_kernel_body(F, idx_hbm, w_hbm, feats_hbm, out_hbm,
                      idx_v0, idx_v1, w_v0, w_v1, rows0, rows1, msgs,
                      gsem0, gsem1, stsem0, stsem1, acc):
    nf = F // 16
    cid = lax.axis_index("c")
    sid = lax.axis_index("s")
    widg = cid * NS + sid
    bufs = ((idx_v0, w_v0, rows0, gsem0, stsem0),
            (idx_v1, w_v1, rows1, gsem1, stsem1))

    zero = jnp.zeros((16,), jnp.float32)

    def zrow(r, _):
        for f in range(nf):
            msgs[r, pl.ds(f * 16, 16)] = zero
        return _

    lax.fori_loop(0, C, zrow, 0, unroll=4)
    for k in range(R_T // C):
        pltpu.sync_copy(msgs, acc.at[pl.ds(sid * R_T + k * C, C)])
    rem = R_T - (R_T // C) * C
    pltpu.sync_copy(msgs.at[pl.ds(0, rem)],
                    acc.at[pl.ds(sid * R_T + R_T - rem, rem)])

    @pl.when(sid == NS - 1)
    def _():
        pltpu.sync_copy(msgs.at[pl.ds(0, R_REM)],
                        acc.at[pl.ds(NS * R_T, R_REM)])

    plsc.subcore_barrier()

    def stage_start(j, b):
        idx_v, w_v, _, _, stsem = bufs[b]
        pltpu.async_copy(idx_hbm.at[:, widg, j], idx_v, stsem)
        pltpu.async_copy(w_hbm.at[pl.ds(widg * E_T + j * C, C)], w_v, stsem)

    def stage_wait(b):
        idx_v, w_v, _, _, stsem = bufs[b]
        pltpu.make_async_copy(idx_hbm.at[:, widg, 0], idx_v, stsem).wait()
        pltpu.make_async_copy(w_hbm.at[pl.ds(0, C)], w_v, stsem).wait()

    def gather_start(b):
        idx_v, _, rows, gsem, _ = bufs[b]
        pltpu.async_copy(feats_hbm.at[idx_v.at[1]], rows, gsem)

    def gather_wait(b):
        idx_v, _, rows, gsem, _ = bufs[b]
        pltpu.make_async_copy(feats_hbm.at[idx_v.at[1]], rows, gsem).wait()

    himask = jnp.full((16,), -65536, jnp.int32)

    def compute_scatter(b):
        idx_v, w_v, rows, _, _ = bufs[b]

        @plsc.parallel_loop(0, C, unroll=8 if F > 64 else 16)
        def edge(e):
            wb = plsc.load_gather(w_v, [jnp.full((16,), e, jnp.int32)])
            for f in range(F // 32):
                v = rows[e, pl.ds(f * 16, 16)]
                lo = plsc.bitcast(v << 16, jnp.float32)
                hi = plsc.bitcast(v & himask, jnp.float32)
                msgs[e, pl.ds(f * 32, 16)] = lo * wb
                msgs[e, pl.ds(f * 32 + 16, 16)] = hi * wb

        pltpu.sync_copy(msgs, acc.at[idx_v.at[0]], add=True)

    stage_start(0, 0)
    stage_start(1, 1)
    stage_wait(0)
    gather_start(0)

    def pair(g, carry):
        j0 = 2 * g
        stage_wait(1)
        gather_start(1)
        gather_wait(0)
        compute_scatter(0)
        stage_start(j0 + 2, 0)
        stage_wait(0)
        gather_start(0)
        gather_wait(1)
        compute_scatter(1)

        @pl.when(j0 + 3 < NCH)
        def _stage_next():
            stage_start(j0 + 3, 1)

        return carry

    lax.fori_loop(0, NCH // 2, pair, 0)

    gather_wait(0)
    compute_scatter(0)

    plsc.subcore_barrier()

    pltpu.sync_copy(acc.at[pl.ds(sid * R_T, R_T)],
                    out_hbm.at[cid, pl.ds(sid * R_T, R_T)])

    @pl.when(sid == NS - 1)
    def _():
        pltpu.sync_copy(acc.at[pl.ds(NS * R_T, R_REM)],
                        out_hbm.at[cid, pl.ds(NS * R_T, R_REM)])


@functools.lru_cache(maxsize=None)
def _make_spmm(F):
    mesh = plsc.VectorSubcoreMesh(core_axis_name="c", subcore_axis_name="s",
                                  num_cores=NC, num_subcores=NS)
    return pl.kernel(
        functools.partial(_spmm_kernel_body, F),
        mesh=mesh,
        out_type=jax.ShapeDtypeStruct((NC, N, F), jnp.float32),
        scratch_types=[
            pltpu.VMEM((2, C), jnp.int32),
            pltpu.VMEM((2, C), jnp.int32),
            pltpu.VMEM((C,), jnp.float32),
            pltpu.VMEM((C,), jnp.float32),
            pltpu.VMEM((C, F // 2), jnp.int32),
            pltpu.VMEM((C, F // 2), jnp.int32),
            pltpu.VMEM((C, F), jnp.float32),
            pltpu.SemaphoreType.DMA,
            pltpu.SemaphoreType.DMA,
            pltpu.SemaphoreType.DMA,
            pltpu.SemaphoreType.DMA,
            pltpu.VMEM_SHARED((N, F), jnp.float32),
        ],
        compiler_params=pltpu.CompilerParams(use_tc_tiling_on_sc=False,
                                             needs_layout_passes=False),
    )


_RB = 1000


def _pack_perm(fb):
    half = fb // 2
    p = np.empty(fb, np.int32)
    for j in range(half):
        p[j] = (j // 16) * 32 + j % 16
        p[half + j] = (j // 16) * 32 + 16 + j % 16
    return p


_PERM_HID = _pack_perm(F_HID)
_PERM_OUT = _pack_perm(F_OUT_PAD)


def _pack_words(y):
    half = y.shape[-1] // 2
    u = lax.bitcast_convert_type(y, jnp.uint32)
    r = u + jnp.uint32(0x7FFF) + ((u >> 16) & jnp.uint32(1))
    lo = r[:, :half] >> 16
    hi = r[:, half:] & jnp.uint32(0xFFFF0000)
    return lax.bitcast_convert_type(lo | hi, jnp.int32)


def _mm_body(x_ref, w_ref, o_ref):
    o_ref[...] = _pack_words(jnp.dot(x_ref[...], w_ref[...],
                                     preferred_element_type=jnp.float32))


def _fuse_body(p_ref, b_ref, w_ref, o_ref):
    h = jnp.maximum(p_ref[0] + p_ref[1] + b_ref[...], 0.0)
    o_ref[...] = _pack_words(jnp.dot(h, w_ref[...],
                                     preferred_element_type=jnp.float32))


def _comb_body(q_ref, b_ref, o_ref):
    s = q_ref[0] + q_ref[1]
    o_ref[...] = s[:, :F_OUT] + b_ref[...]


def _mm(x, w):
    return pl.pallas_call(
        _mm_body,
        grid=(N // _RB,),
        in_specs=[pl.BlockSpec((_RB, F_IN), lambda i: (i, 0)),
                  pl.BlockSpec((F_IN, F_HID), lambda i: (0, 0))],
        out_specs=pl.BlockSpec((_RB, F_HID // 2), lambda i: (i, 0)),
        out_shape=jax.ShapeDtypeStruct((N, F_HID // 2), jnp.int32),
    )(x, w)


def _fuse(p, b1, w2p):
    return pl.pallas_call(
        _fuse_body,
        grid=(N // _RB,),
        in_specs=[pl.BlockSpec((NC, _RB, F_HID), lambda i: (0, i, 0)),
                  pl.BlockSpec((1, F_HID), lambda i: (0, 0)),
                  pl.BlockSpec((F_HID, F_OUT_PAD), lambda i: (0, 0))],
        out_specs=pl.BlockSpec((_RB, F_OUT_PAD // 2), lambda i: (i, 0)),
        out_shape=jax.ShapeDtypeStruct((N, F_OUT_PAD // 2), jnp.int32),
    )(p, b1, w2p)


def _combine(q, b2):
    return pl.pallas_call(
        _comb_body,
        grid=(N // _RB,),
        in_specs=[pl.BlockSpec((NC, _RB, F_OUT_PAD), lambda i: (0, i, 0)),
                  pl.BlockSpec((1, F_OUT), lambda i: (0, 0))],
        out_specs=pl.BlockSpec((_RB, F_OUT), lambda i: (i, 0)),
        out_shape=jax.ShapeDtypeStruct((N, F_OUT), jnp.float32),
    )(q, b2)


def kernel(x, edge_index, edge_weight, W1, b1, W2, b2):
    idx = edge_index.astype(jnp.int32).reshape(2, NW, NCH, C)
    w = edge_weight.astype(jnp.float32)
    w2p = jnp.pad(W2, ((0, 0), (0, F_OUT_PAD - F_OUT)))[:, _PERM_OUT]

    xw1 = _mm(x, W1[:, _PERM_HID])
    p = _make_spmm(F_HID)(idx, w, xw1)
    hw2 = _fuse(p, b1.reshape(1, F_HID), w2p)
    q = _make_spmm(F_OUT_PAD)(idx, w, hw2)
    return _combine(q, b2.reshape(1, F_OUT))


# speedup vs baseline: 1.0457x; 1.0029x over previous
import functools

import jax
import jax.numpy as jnp
import numpy as np
from jax import lax
from jax.experimental import pallas as pl
from jax.experimental.pallas import tpu as pltpu
from jax.experimental.pallas import tpu_sc as plsc

N = 10000
E = 320000
F_IN = 128
F_HID = 128
F_OUT = 40
F_OUT_PAD = 64

NC = 2
NS = 16
NW = NC * NS
E_T = E // NW
C = 80
NCH = E_T // C
R_T = 624
R_REM = N - NS * R_T


def _spmm_kernel_body(F, idx_hbm, w_hbm, feats_hbm, out_hbm,
                      idx_v0, idx_v1, w_v0, w_v1, rows0, rows1, msgs,
                      gsem0, gsem1, stsem0, stsem1, acc):
    nf = F // 16
    cid = lax.axis_index("c")
    sid = lax.axis_index("s")
    widg = cid * NS + sid
    bufs = ((idx_v0, w_v0, rows0, gsem0, stsem0),
            (idx_v1, w_v1, rows1, gsem1, stsem1))

    zero = jnp.zeros((16,), jnp.float32)

    def zrow(r, _):
        for f in range(nf):
            msgs[r, pl.ds(f * 16, 16)] = zero
        return _

    lax.fori_loop(0, C, zrow, 0, unroll=4)
    for k in range(R_T // C):
        pltpu.sync_copy(msgs, acc.at[pl.ds(sid * R_T + k * C, C)])
    rem = R_T - (R_T // C) * C
    pltpu.sync_copy(msgs.at[pl.ds(0, rem)],
                    acc.at[pl.ds(sid * R_T + R_T - rem, rem)])

    @pl.when(sid == NS - 1)
    def _():
        pltpu.sync_copy(msgs.at[pl.ds(0, R_REM)],
                        acc.at[pl.ds(NS * R_T, R_REM)])

    plsc.subcore_barrier()

    def stage_start(j, b):
        idx_v, w_v, _, _, stsem = bufs[b]
        pltpu.async_copy(idx_hbm.at[:, widg, j], idx_v, stsem)
        pltpu.async_copy(w_hbm.at[pl.ds(widg * E_T + j * C, C)], w_v, stsem)

    def stage_wait(b):
        idx_v, w_v, _, _, stsem = bufs[b]
        pltpu.make_async_copy(idx_hbm.at[:, widg, 0], idx_v, stsem).wait()
        pltpu.make_async_copy(w_hbm.at[pl.ds(0, C)], w_v, stsem).wait()

    def gather_start(b):
        idx_v, _, rows, gsem, _ = bufs[b]
        pltpu.async_copy(feats_hbm.at[idx_v.at[1]], rows, gsem)

    def gather_wait(b):
        idx_v, _, rows, gsem, _ = bufs[b]
        pltpu.make_async_copy(feats_hbm.at[idx_v.at[1]], rows, gsem).wait()

    himask = jnp.full((16,), -65536, jnp.int32)

    def compute_scatter(b):
        idx_v, w_v, rows, _, _ = bufs[b]

        @plsc.parallel_loop(0, C, unroll=8)
        def edge(e):
            wb = plsc.load_gather(w_v, [jnp.full((16,), e, jnp.int32)])
            for f in range(F // 32):
                v = rows[e, pl.ds(f * 16, 16)]
                lo = plsc.bitcast(v << 16, jnp.float32)
                hi = plsc.bitcast(v & himask, jnp.float32)
                msgs[e, pl.ds(f * 32, 16)] = lo * wb
                msgs[e, pl.ds(f * 32 + 16, 16)] = hi * wb

        pltpu.sync_copy(msgs, acc.at[idx_v.at[0]], add=True)

    stage_start(0, 0)
    stage_start(1, 1)
    stage_wait(0)
    gather_start(0)

    def pair(g, carry):
        j0 = 2 * g
        stage_wait(1)
        gather_start(1)
        gather_wait(0)
        compute_scatter(0)
        stage_start(j0 + 2, 0)
        stage_wait(0)
        gather_start(0)
        gather_wait(1)
        compute_scatter(1)

        @pl.when(j0 + 3 < NCH)
        def _stage_next():
            stage_start(j0 + 3, 1)

        return carry

    lax.fori_loop(0, NCH // 2, pair, 0)

    gather_wait(0)
    compute_scatter(0)

    plsc.subcore_barrier()

    pltpu.sync_copy(acc.at[pl.ds(sid * R_T, R_T)],
                    out_hbm.at[cid, pl.ds(sid * R_T, R_T)])

    @pl.when(sid == NS - 1)
    def _():
        pltpu.sync_copy(acc.at[pl.ds(NS * R_T, R_REM)],
                        out_hbm.at[cid, pl.ds(NS * R_T, R_REM)])


@functools.lru_cache(maxsize=None)
def _make_spmm(F):
    mesh = plsc.VectorSubcoreMesh(core_axis_name="c", subcore_axis_name="s",
                                  num_cores=NC, num_subcores=NS)
    return pl.kernel(
        functools.partial(_spmm_kernel_body, F),
        mesh=mesh,
        out_type=jax.ShapeDtypeStruct((NC, N, F), jnp.float32),
        scratch_types=[
            pltpu.VMEM((2, C), jnp.int32),
            pltpu.VMEM((2, C), jnp.int32),
            pltpu.VMEM((C,), jnp.float32),
            pltpu.VMEM((C,), jnp.float32),
            pltpu.VMEM((C, F // 2), jnp.int32),
            pltpu.VMEM((C, F // 2), jnp.int32),
            pltpu.VMEM((C, F), jnp.float32),
            pltpu.SemaphoreType.DMA,
            pltpu.SemaphoreType.DMA,
            pltpu.SemaphoreType.DMA,
            pltpu.SemaphoreType.DMA,
            pltpu.VMEM_SHARED((N, F), jnp.float32),
        ],
        compiler_params=pltpu.CompilerParams(use_tc_tiling_on_sc=False,
                                             needs_layout_passes=False),
    )


_RB = 1000


def _pack_perm(fb):
    half = fb // 2
    p = np.empty(fb, np.int32)
    for j in range(half):
        p[j] = (j // 16) * 32 + j % 16
        p[half + j] = (j // 16) * 32 + 16 + j % 16
    return p


_PERM_HID = _pack_perm(F_HID)
_PERM_OUT = _pack_perm(F_OUT_PAD)


def _pack_words(y):
    half = y.shape[-1] // 2
    u = lax.bitcast_convert_type(y, jnp.uint32)
    r = u + jnp.uint32(0x7FFF) + ((u >> 16) & jnp.uint32(1))
    lo = r[:, :half] >> 16
    hi = r[:, half:] & jnp.uint32(0xFFFF0000)
    return lax.bitcast_convert_type(lo | hi, jnp.int32)


def _mm_body(x_ref, w_ref, o_ref):
    o_ref[...] = _pack_words(jnp.dot(x_ref[...], w_ref[...],
                                     preferred_element_type=jnp.float32))


def _fuse_body(p_ref, b_ref, w_ref, o_ref):
    h = jnp.maximum(p_ref[0] + p_ref[1] + b_ref[...], 0.0)
    o_ref[...] = _pack_words(jnp.dot(h, w_ref[...],
                                     preferred_element_type=jnp.float32))


def _comb_body(q_ref, b_ref, o_ref):
    s = q_ref[0] + q_ref[1]
    o_ref[...] = s[:, :F_OUT] + b_ref[...]


def _mm(x, w):
    return pl.pallas_call(
        _mm_body,
        grid=(N // _RB,),
        in_specs=[pl.BlockSpec((_RB, F_IN), lambda i: (i, 0)),
                  pl.BlockSpec((F_IN, F_HID), lambda i: (0, 0))],
        out_specs=pl.BlockSpec((_RB, F_HID // 2), lambda i: (i, 0)),
        out_shape=jax.ShapeDtypeStruct((N, F_HID // 2), jnp.int32),
    )(x, w)


def _fuse(p, b1, w2p):
    return pl.pallas_call(
        _fuse_body,
        grid=(N // _RB,),
        in_specs=[pl.BlockSpec((NC, _RB, F_HID), lambda i: (0, i, 0)),
                  pl.BlockSpec((1, F_HID), lambda i: (0, 0)),
                  pl.BlockSpec((F_HID, F_OUT_PAD), lambda i: (0, 0))],
        out_specs=pl.BlockSpec((_RB, F_OUT_PAD // 2), lambda i: (i, 0)),
        out_shape=jax.ShapeDtypeStruct((N, F_OUT_PAD // 2), jnp.int32),
    )(p, b1, w2p)


def _combine(q, b2):
    return pl.pallas_call(
        _comb_body,
        grid=(N // _RB,),
        in_specs=[pl.BlockSpec((NC, _RB, F_OUT_PAD), lambda i: (0, i, 0)),
                  pl.BlockSpec((1, F_OUT), lambda i: (0, 0))],
        out_specs=pl.BlockSpec((_RB, F_OUT), lambda i: (i, 0)),
        out_shape=jax.ShapeDtypeStruct((N, F_OUT), jnp.float32),
    )(q, b2)


def kernel(x, edge_index, edge_weight, W1, b1, W2, b2):
    idx = edge_index.astype(jnp.int32).reshape(2, NW, NCH, C)
    w = edge_weight.astype(jnp.float32)
    w2p = jnp.pad(W2, ((0, 0), (0, F_OUT_PAD - F_OUT)))[:, _PERM_OUT]

    xw1 = _mm(x, W1[:, _PERM_HID])
    p = _make_spmm(F_HID)(idx, w, xw1)
    hw2 = _fuse(p, b1.reshape(1, F_HID), w2p)
    q = _make_spmm(F_OUT_PAD)(idx, w, hw2)
    return _combine(q, b2.reshape(1, F_OUT))
